# Initial kernel scaffold; baseline (speedup 1.0000x reference)
import jax
import jax.numpy as jnp
from jax.experimental import pallas as pl


def kernel(x, edge_index, edge_attr, W1, We1, as1, ad1, ae1, b1, W2, We2, as2, ad2, ae2, b2, W3, We3, as3, ad3, ae3, b3, Wc, bc):
    raise NotImplementedError("write your pallas kernel here")


# speedup vs baseline: 1.5361x; 1.5361x over previous
import jax
import jax.numpy as jnp
from jax.experimental import pallas as pl
from jax.experimental.pallas import tpu as pltpu

N = 10000


def _gat(x, src, dst, ea, W, We, a_s, a_d, a_e, b):
    h = x @ W
    ve = We @ a_e
    a_i = h @ a_s
    a_j = h @ a_d
    a_eg = ea @ ve
    alpha = jax.nn.leaky_relu(a_i[src] + a_j[dst] + a_eg, negative_slope=0.2)
    ex = jnp.exp(alpha)
    den = jax.ops.segment_sum(ex, dst, num_segments=N)
    out = jax.ops.segment_sum(ex[:, None] * h[src], dst, num_segments=N)
    return out / (den[:, None] + 1e-16) + b


def _final_kernel(h_ref, wc_ref, bc_ref, o_ref):
    logits = jnp.dot(h_ref[...], wc_ref[...], preferred_element_type=jnp.float32)
    logits = logits + bc_ref[...][None, :]
    m = jnp.max(logits, axis=0, keepdims=True)
    lse = m + jnp.log(jnp.sum(jnp.exp(logits - m), axis=0, keepdims=True))
    o_ref[...] = logits - lse


def kernel(x, edge_index, edge_attr, W1, We1, as1, ad1, ae1, b1, W2, We2, as2, ad2, ae2, b2, W3, We3, as3, ad3, ae3, b3, Wc, bc):
    src = edge_index[0]
    dst = edge_index[1]
    h = jax.nn.relu(_gat(x, src, dst, edge_attr, W1, We1, as1, ad1, ae1, b1))
    h = jax.nn.relu(_gat(h, src, dst, edge_attr, W2, We2, as2, ad2, ae2, b2))
    h = jax.nn.relu(_gat(h, src, dst, edge_attr, W3, We3, as3, ad3, ae3, b3))
    out = pl.pallas_call(
        _final_kernel,
        out_shape=jax.ShapeDtypeStruct((N, 256), jnp.float32),
    )(h, Wc, bc)
    return out


# speedup vs baseline: 4.5216x; 2.9437x over previous
import functools

import jax
import jax.numpy as jnp
from jax import lax
from jax.experimental import pallas as pl
from jax.experimental.pallas import tpu as pltpu
from jax.experimental.pallas import tpu_sc as plsc

N = 10000
E = 160000
NP = 10240
NB = 64
R = 160
CAP = 8192
CH = 2000
BA = 256
PACK = 16384

_f32 = jnp.float32
_i32 = jnp.int32


def _prep_body(ea_ref, we1_ref, ae1_ref, we2_ref, ae2_ref, we3_ref, ae3_ref,
               g1_ref, g2_ref, g3_ref):
    ea = ea_ref[...]
    for we_ref, ae_ref, g_ref in ((we1_ref, ae1_ref, g1_ref),
                                  (we2_ref, ae2_ref, g2_ref),
                                  (we3_ref, ae3_ref, g3_ref)):
        v = jnp.sum(we_ref[...] * ae_ref[...], axis=1)
        g_ref[...] = (ea @ v)[None, None, :]


def _edge_scalars(edge_attr, We1, ae1, We2, ae2, We3, ae3):
    blk = 3200
    grid = E // blk
    return pl.pallas_call(
        _prep_body,
        grid=(grid,),
        in_specs=[
            pl.BlockSpec((blk, 16), lambda i: (i, 0)),
            pl.BlockSpec((16, 16), lambda i: (0, 0)),
            pl.BlockSpec((16, 16), lambda i: (0, 0)),
            pl.BlockSpec((16, 256), lambda i: (0, 0)),
            pl.BlockSpec((16, 256), lambda i: (0, 0)),
            pl.BlockSpec((16, 64), lambda i: (0, 0)),
            pl.BlockSpec((16, 64), lambda i: (0, 0)),
        ],
        out_specs=[pl.BlockSpec((1, 1, blk), lambda i: (i, 0, 0))] * 3,
        out_shape=[jax.ShapeDtypeStruct((grid, 1, blk), _f32)] * 3,
    )(edge_attr, We1, jnp.broadcast_to(ae1[None, :], (16, ae1.shape[0])),
      We2, jnp.broadcast_to(ae2[None, :], (16, ae2.shape[0])),
      We3, jnp.broadcast_to(ae3[None, :], (16, ae3.shape[0])))


def _layer_body(fin, fout, fpad, normalize, prev_ref, w_ref, as_ref, ad_ref,
                b_ref, h_ref, ai_ref, aj_ref):
    pb = prev_ref[...]
    if normalize:
        xin = pb[:, :fin] / (pb[:, fin:fin + 1] + 1e-16) + b_ref[...]
        xin = jnp.maximum(xin, 0.0)
    else:
        xin = pb
    h = jnp.dot(xin, w_ref[...], preferred_element_type=_f32)
    ai_ref[...] = h @ as_ref[...][0]
    aj_ref[...] = h @ ad_ref[...][0]
    br = h.shape[0]
    h_ref[...] = jnp.concatenate(
        [h, jnp.ones((br, 1), _f32), jnp.zeros((br, fpad - fout - 1), _f32)],
        axis=1)


def _layer_dense(prev, W, a_s, a_d, b_in, fin, fout, fpad, normalize):
    br = 1024
    grid = NP // br
    fp = prev.shape[1]
    return pl.pallas_call(
        functools.partial(_layer_body, fin, fout, fpad, normalize),
        grid=(grid,),
        in_specs=[
            pl.BlockSpec((br, fp), lambda i: (i, 0)),
            pl.BlockSpec((fin, fout), lambda i: (0, 0)),
            pl.BlockSpec((1, fout), lambda i: (0, 0)),
            pl.BlockSpec((1, fout), lambda i: (0, 0)),
            pl.BlockSpec((1, fin), lambda i: (0, 0)),
        ],
        out_specs=[
            pl.BlockSpec((br, fpad), lambda i: (i, 0)),
            pl.BlockSpec((br,), lambda i: (i,)),
            pl.BlockSpec((br,), lambda i: (i,)),
        ],
        out_shape=[
            jax.ShapeDtypeStruct((NP, fpad), _f32),
            jax.ShapeDtypeStruct((NP,), _f32),
            jax.ShapeDtypeStruct((NP,), _f32),
        ],
    )(prev, W, a_s[None, :], a_d[None, :], b_in[None, :])


def _final_body(prev_ref, b_ref, wc_ref, bc_ref, o_ref):
    pb = prev_ref[...]
    xin = pb[:, :64] / (pb[:, 64:65] + 1e-16) + b_ref[...]
    xin = jnp.maximum(xin, 0.0)
    logits = jnp.dot(xin, wc_ref[...], preferred_element_type=_f32)
    logits = logits + bc_ref[...]
    rows = lax.broadcasted_iota(_i32, logits.shape, 0)
    valid = rows < N
    neg = jnp.float32(-1e30)
    m = jnp.max(jnp.where(valid, logits, neg), axis=0, keepdims=True)
    se = jnp.sum(jnp.where(valid, jnp.exp(logits - m), 0.0), axis=0,
                 keepdims=True)
    o_ref[...] = logits - m - jnp.log(se)


def _final_dense(prev, b3, Wc, bc):
    return pl.pallas_call(
        _final_body,
        out_shape=jax.ShapeDtypeStruct((NP, 256), _f32),
    )(prev, b3[None, :], Wc, bc[None, :])


@functools.cache
def _mesh():
    return plsc.VectorSubcoreMesh(core_axis_name="c", subcore_axis_name="s")


def _wid():
    return lax.axis_index("s") * 2 + lax.axis_index("c")


def _build_body(src_hbm, dst_hbm, g1_hbm, g2_hbm, g3_hbm,
                cnt_hbm, pedg_hbm, pg1_hbm, pg2_hbm, pg3_hbm,
                st_src, st_dst, st_g1, st_g2, st_g3,
                b_edg, b_g1, b_g2, b_g3, cnt_v, tmp, sem):
    wid = _wid()
    iota = lax.broadcasted_iota(_i32, (16,), 0)
    tmp[pl.ds(0, 16)] = jnp.zeros((16,), _i32)

    def prefix16(v):
        for k in (1, 2, 4, 8):
            tmp[pl.ds(16, 16)] = v
            v = v + plsc.load_gather(tmp, [iota + (16 - k)])
        return v

    def chunk(ci, cnts):
        base = ci * CH
        pltpu.async_copy(src_hbm.at[pl.ds(base, CH)], st_src, sem).wait()
        pltpu.async_copy(dst_hbm.at[pl.ds(base, CH)], st_dst, sem).wait()
        pltpu.async_copy(g1_hbm.at[pl.ds(base, CH)], st_g1, sem).wait()
        pltpu.async_copy(g2_hbm.at[pl.ds(base, CH)], st_g2, sem).wait()
        pltpu.async_copy(g3_hbm.at[pl.ds(base, CH)], st_g3, sem).wait()

        def vec(k, cnts):
            d = st_dst[pl.ds(k * 16, 16)]
            s = st_src[pl.ds(k * 16, 16)]
            new = []
            for p in range(2):
                lo = (wid * 2 + p) * R
                cnt = cnts[p]
                dloc = d - lo
                msk = (d >= lo) & (d < lo + R)
                rec = s + dloc * PACK
                pref = prefix16(jnp.where(msk, 1, 0).astype(_i32))
                pos = cnt + pref - 1 + (p * CAP)
                plsc.store_scatter(b_edg, [pos], rec, mask=msk)
                plsc.store_scatter(b_g1, [pos],
                                   st_g1[pl.ds(k * 16, 16)], mask=msk)
                plsc.store_scatter(b_g2, [pos],
                                   st_g2[pl.ds(k * 16, 16)], mask=msk)
                plsc.store_scatter(b_g3, [pos],
                                   st_g3[pl.ds(k * 16, 16)], mask=msk)
                new.append(jnp.minimum(cnt + pref[15], CAP - BA - 16))
            return tuple(new)

        return lax.fori_loop(0, CH // 16, vec, cnts)

    cnts = lax.fori_loop(0, E // CH, chunk, (jnp.int32(0), jnp.int32(0)))

    for p in range(2):
        cnt = cnts[p]
        for i in range(BA // 16):
            b_edg[pl.ds(p * CAP + cnt + i * 16, 16)] = jnp.full(
                (16,), R * PACK, _i32)
            b_g1[pl.ds(p * CAP + cnt + i * 16, 16)] = jnp.zeros((16,), _f32)
            b_g2[pl.ds(p * CAP + cnt + i * 16, 16)] = jnp.zeros((16,), _f32)
            b_g3[pl.ds(p * CAP + cnt + i * 16, 16)] = jnp.zeros((16,), _f32)
        cnt = jnp.bitwise_and(cnt + BA - 1, -BA)
        bid = wid * 2 + p
        cnt_v[...] = jnp.full((16,), cnt, _i32)
        pltpu.async_copy(cnt_v, cnt_hbm.at[pl.ds(bid * 16, 16)], sem).wait()
        pltpu.async_copy(b_edg.at[pl.ds(p * CAP, CAP)],
                         pedg_hbm.at[pl.ds(bid * CAP, CAP)], sem).wait()
        pltpu.async_copy(b_g1.at[pl.ds(p * CAP, CAP)],
                         pg1_hbm.at[pl.ds(bid * CAP, CAP)], sem).wait()
        pltpu.async_copy(b_g2.at[pl.ds(p * CAP, CAP)],
                         pg2_hbm.at[pl.ds(bid * CAP, CAP)], sem).wait()
        pltpu.async_copy(b_g3.at[pl.ds(p * CAP, CAP)],
                         pg3_hbm.at[pl.ds(bid * CAP, CAP)], sem).wait()


@functools.cache
def _make_build():
    return pl.kernel(
        _build_body,
        out_type=[
            jax.ShapeDtypeStruct((NB * 16,), _i32),
            jax.ShapeDtypeStruct((NB * CAP,), _i32),
            jax.ShapeDtypeStruct((NB * CAP,), _f32),
            jax.ShapeDtypeStruct((NB * CAP,), _f32),
            jax.ShapeDtypeStruct((NB * CAP,), _f32),
        ],
        mesh=_mesh(),
        compiler_params=pltpu.CompilerParams(needs_layout_passes=False),
        scratch_types=[
            pltpu.VMEM((CH,), _i32), pltpu.VMEM((CH,), _i32),
            pltpu.VMEM((CH,), _f32), pltpu.VMEM((CH,), _f32),
            pltpu.VMEM((CH,), _f32),
            pltpu.VMEM((2 * CAP,), _i32),
            pltpu.VMEM((2 * CAP,), _f32), pltpu.VMEM((2 * CAP,), _f32),
            pltpu.VMEM((2 * CAP,), _f32),
            pltpu.VMEM((16,), _i32),
            pltpu.VMEM((32,), _i32),
            pltpu.SemaphoreType.DMA,
        ],
    )


def _agg_body(fpad, nacc, rb, level, h_hbm, ai_hbm, aj_hbm, cnt_hbm, pedg_hbm,
              pg_hbm, out_hbm, ai_loc, aj_loc, edg_blk, g_blk, w_blk, acc,
              src_sb0, src_sb1, cnt_v, rows, sem, gsem0, gsem1):
    wid = _wid()
    nsb = BA // rb
    src_sbs = (src_sb0, src_sb1)
    gsems = (gsem0, gsem1)

    pltpu.async_copy(ai_hbm, ai_loc, sem).wait()

    for p in range(2):
        bid = wid * 2 + p
        lo = bid * R
        pltpu.async_copy(aj_hbm.at[pl.ds(lo, R)], aj_loc.at[pl.ds(0, R)],
                         sem).wait()
        aj_loc[pl.ds(R, 16)] = jnp.zeros((16,), _f32)

        def zrow(r, _):
            for c in range(fpad // 16):
                acc[r, pl.ds(c * 16, 16)] = jnp.zeros((16,), _f32)
            return 0

        lax.fori_loop(0, R + 8, zrow, 0)

        pltpu.async_copy(cnt_hbm.at[pl.ds(bid * 16, 16)], cnt_v, sem).wait()
        cnt_s = cnt_v[...][0]

        def block(bi, _):
            @pl.when(bi * BA < cnt_s)
            def _process():
                _agg_block(bi)
            return 0

        if level < 2:
            def block(bi, _):
                return 0

        def _agg_block(bi):
            base = bid * CAP + bi * BA
            pltpu.async_copy(pedg_hbm.at[pl.ds(base, BA)],
                             edg_blk.at[pl.ds(0, BA)], sem).wait()
            pltpu.async_copy(pg_hbm.at[pl.ds(base, BA)], g_blk, sem).wait()
            if level < 3:
                return

            def wvec(k, _):
                rec = edg_blk[pl.ds(k * 16, 16)]
                s16 = jnp.minimum(jnp.bitwise_and(rec, PACK - 1), NP - 1)
                d16 = jnp.minimum(
                    jnp.bitwise_and(lax.shift_right_logical(rec, 14),
                                    2 * PACK - 1), R)
                if level >= 4:
                    a = (plsc.load_gather(ai_loc, [s16])
                         + plsc.load_gather(aj_loc, [d16])
                         + g_blk[pl.ds(k * 16, 16)])
                else:
                    a = (g_blk[pl.ds(k * 16, 16)]
                         + 1e-6 * (s16 + d16).astype(_f32))
                a = jnp.where(a > 0, a, 0.2 * a)
                w_blk[pl.ds(k * 16, 16)] = jnp.exp(a)
                return 0

            lax.fori_loop(0, BA // 16, wvec, 0)
            if level < 5:
                return

            def stage(sb, par):
                for t in range(rb // 16):
                    rec = edg_blk[pl.ds(sb * rb + t * 16, 16)]
                    src_sbs[par][pl.ds(t * 16, 16)] = jnp.minimum(
                        jnp.bitwise_and(rec, PACK - 1), NP - 1)
                pltpu.async_copy(h_hbm.at[src_sbs[par]], rows.at[par],
                                 gsems[par])

            stage(0, 0)
            if nsb > 1:
                stage(1, 1)
            for sb in range(nsb):
                par = sb % 2
                pltpu.make_async_copy(h_hbm.at[src_sbs[par]], rows.at[par],
                                      gsems[par]).wait()

                def edge(j, _):
                    off = sb * rb + j
                    rec = edg_blk[pl.ds(off, 16)][0]
                    w = w_blk[pl.ds(off, 16)][0]
                    dloc = jnp.minimum(
                        jnp.bitwise_and(lax.shift_right_logical(rec, 14),
                                        2 * PACK - 1), R)
                    for c in range(nacc // 16):
                        sl = pl.ds(c * 16, 16)
                        acc[dloc, sl] = acc[dloc, sl] + w * rows[par, j, sl]
                    return 0

                if level >= 6:
                    lax.fori_loop(0, rb, edge, 0)
                if sb + 2 < nsb:
                    stage(sb + 2, par)
            return 0

        lax.fori_loop(0, CAP // BA, block, 0)
        pltpu.async_copy(acc.at[pl.ds(0, R)], out_hbm.at[pl.ds(lo, R)],
                         sem).wait()


@functools.cache
def _make_agg(fout, fpad, rb, level=4):
    nacc = fout + 16
    return pl.kernel(
        functools.partial(_agg_body, fpad, nacc, rb, level),
        out_type=jax.ShapeDtypeStruct((NP, fpad), _f32),
        mesh=_mesh(),
        compiler_params=pltpu.CompilerParams(needs_layout_passes=False),
        scratch_types=[
            pltpu.VMEM((NP,), _f32),
            pltpu.VMEM((R + 16,), _f32),
            pltpu.VMEM((BA + 16,), _i32),
            pltpu.VMEM((BA,), _f32),
            pltpu.VMEM((BA + 16,), _f32),
            pltpu.VMEM((R + 8, fpad), _f32),
            pltpu.VMEM((rb,), _i32),
            pltpu.VMEM((rb,), _i32),
            pltpu.VMEM((16,), _i32),
            pltpu.VMEM((2, rb, fpad), _f32),
            pltpu.SemaphoreType.DMA,
            pltpu.SemaphoreType.DMA,
            pltpu.SemaphoreType.DMA,
        ],
    )


def kernel(x, edge_index, edge_attr, W1, We1, as1, ad1, ae1, b1, W2, We2,
           as2, ad2, ae2, b2, W3, We3, as3, ad3, ae3, b3, Wc, bc):
    src = edge_index[0]
    dst = edge_index[1]
    xp = jnp.pad(x, ((0, NP - N), (0, 0)))

    g1, g2, g3 = _edge_scalars(edge_attr, We1, ae1, We2, ae2, We3, ae3)
    g1, g2, g3 = g1.reshape(E), g2.reshape(E), g3.reshape(E)
    cnt, pedg, pg1, pg2, pg3 = _make_build()(src, dst, g1, g2, g3)

    h1, ai1, aj1 = _layer_dense(xp, W1, as1, ad1, jnp.zeros((256,), _f32),
                                256, 16, 128, False)
    o1 = _make_agg(16, 128, 64, 6)(h1, ai1, aj1, cnt, pedg, pg1)

    h2, ai2, aj2 = _layer_dense(o1, W2, as2, ad2, b1, 16, 256, 384, True)
    o2 = _make_agg(256, 384, 16, 6)(h2, ai2, aj2, cnt, pedg, pg2)

    h3, ai3, aj3 = _layer_dense(o2, W3, as3, ad3, b2, 256, 64, 128, True)
    o3 = _make_agg(64, 128, 32, 6)(h3, ai3, aj3, cnt, pedg, pg3)

    out = _final_dense(o3, b3, Wc, bc)
    return out[:N]
